# baseline (device time: 48572 ns/iter reference)
import jax
import jax.numpy as jnp
from jax import lax
from jax.experimental import pallas as pl
from jax.experimental.pallas import tpu as pltpu

N_DEV = 4


def kernel(x, Wq, K_ext, V_ext, Wo):
    B_loc, Sq, E = x.shape
    _, wq_cols = Wq.shape
    Bg, Skv, Hq, Dh = K_ext.shape
    H_loc = wq_cols // Dh
    Eo = Wo.shape[1]

    my_pos = lax.axis_index("i")

    xf = x.reshape(B_loc * Sq, E)
    Kb = lax.dynamic_slice_in_dim(K_ext, B_loc * my_pos, B_loc, axis=0)
    Vb = lax.dynamic_slice_in_dim(V_ext, B_loc * my_pos, B_loc, axis=0)
    Kb = jnp.transpose(Kb, (2, 0, 1, 3)).reshape(Hq * B_loc * Sq, Dh)
    Vb = jnp.transpose(Vb, (2, 0, 1, 3)).reshape(Hq * B_loc * Sq, Dh)

    def body(x_ref, wq_ref, k_ref, v_ref, wo_ref, out_ref,
             wq_comm, wo_comm, ctx_ref,
             wq_send, wq_recv, wo_send, wo_recv):
        my = lax.axis_index("i")
        left = lax.rem(my + N_DEV - 1, N_DEV)
        right = lax.rem(my + 1, N_DEV)

        barrier = pltpu.get_barrier_semaphore()
        for nbr in (left, right):
            pl.semaphore_signal(
                barrier, inc=1,
                device_id=(nbr,), device_id_type=pl.DeviceIdType.MESH,
            )
        pl.semaphore_wait(barrier, 2)

        wq_comm[0, :, :] = wq_ref[:, :]
        wo_comm[0, :, :] = wo_ref[:, :]

        def compute_block(h):
            j = lax.rem(my + N_DEV - h, N_DEV)
            q_all = jnp.dot(x_ref[:, :], wq_comm[h],
                            preferred_element_type=jnp.float32)
            for b in range(B_loc):
                for hh in range(H_loc):
                    q = q_all[b * Sq:(b + 1) * Sq, hh * Dh:(hh + 1) * Dh]
                    off = ((j * H_loc + hh) * B_loc + b) * Sq
                    k = k_ref[pl.ds(off, Skv), :]
                    v = v_ref[pl.ds(off, Skv), :]
                    s = lax.dot_general(
                        q, k, (((1,), (1,)), ((), ())),
                        preferred_element_type=jnp.float32) * 0.125
                    m = jnp.max(s, axis=1, keepdims=True)
                    w = jnp.exp(s - m)
                    w = w / jnp.sum(w, axis=1, keepdims=True)
                    ctx_ref[b * Sq:(b + 1) * Sq, hh * Dh:(hh + 1) * Dh] = (
                        jnp.dot(w, v, preferred_element_type=jnp.float32))
            part = jnp.dot(ctx_ref[:, :], wo_comm[h],
                           preferred_element_type=jnp.float32)
            if h == 0:
                out_ref[:, :] = part
            else:
                out_ref[:, :] += part

        for h in range(N_DEV - 1):
            rq = pltpu.make_async_remote_copy(
                src_ref=wq_comm.at[h], dst_ref=wq_comm.at[h + 1],
                send_sem=wq_send.at[h], recv_sem=wq_recv.at[h],
                device_id=(right,), device_id_type=pl.DeviceIdType.MESH,
            )
            ro = pltpu.make_async_remote_copy(
                src_ref=wo_comm.at[h], dst_ref=wo_comm.at[h + 1],
                send_sem=wo_send.at[h], recv_sem=wo_recv.at[h],
                device_id=(right,), device_id_type=pl.DeviceIdType.MESH,
            )
            rq.start()
            ro.start()
            compute_block(h)
            rq.wait()
            ro.wait()
        compute_block(N_DEV - 1)

    out_flat = pl.pallas_call(
        body,
        out_shape=jax.ShapeDtypeStruct((B_loc * Sq, Eo), jnp.float32),
        in_specs=[pl.BlockSpec(memory_space=pltpu.VMEM)] * 5,
        out_specs=pl.BlockSpec(memory_space=pltpu.VMEM),
        scratch_shapes=[
            pltpu.VMEM((N_DEV, E, wq_cols), jnp.float32),
            pltpu.VMEM((N_DEV, wq_cols, Eo), jnp.float32),
            pltpu.VMEM((B_loc * Sq, wq_cols), jnp.float32),
            pltpu.SemaphoreType.DMA((N_DEV - 1,)),
            pltpu.SemaphoreType.DMA((N_DEV - 1,)),
            pltpu.SemaphoreType.DMA((N_DEV - 1,)),
            pltpu.SemaphoreType.DMA((N_DEV - 1,)),
        ],
        compiler_params=pltpu.CompilerParams(collective_id=0),
    )(xf, Wq, Kb, Vb, Wo)

    return out_flat.reshape(B_loc, Sq, Eo)


# device time: 26863 ns/iter; 1.8081x vs baseline; 1.8081x over previous
import jax
import jax.numpy as jnp
from jax import lax
from jax.experimental import pallas as pl
from jax.experimental.pallas import tpu as pltpu

N_DEV = 4


def kernel(x, Wq, K_ext, V_ext, Wo):
    B_loc, Sq, E = x.shape
    _, wq_cols = Wq.shape
    Bg, Skv, Hq, Dh = K_ext.shape
    H_loc = wq_cols // Dh
    Eo = Wo.shape[1]

    my_pos = lax.axis_index("i")

    xf = x.reshape(B_loc * Sq, E).astype(jnp.bfloat16)
    Kb = lax.dynamic_slice_in_dim(K_ext, B_loc * my_pos, B_loc, axis=0)
    Vb = lax.dynamic_slice_in_dim(V_ext, B_loc * my_pos, B_loc, axis=0)
    Kb = jnp.transpose(Kb, (2, 0, 1, 3)).reshape(Hq * B_loc * Sq, Dh)
    Vb = jnp.transpose(Vb, (2, 0, 1, 3)).reshape(Hq * B_loc * Sq, Dh)
    Kb = Kb.astype(jnp.bfloat16)
    Vb = Vb.astype(jnp.bfloat16)
    Wq16 = Wq.astype(jnp.bfloat16)
    Wo16 = Wo.astype(jnp.bfloat16)

    hq2 = (E // 2)
    ho2 = (wq_cols // 2)

    def body(x_ref, wq_ref, k_ref, v_ref, wo_ref, out_ref,
             wqg, wog, ctx_ref, send_sems, recv_sems):
        my = lax.axis_index("i")
        left = lax.rem(my + N_DEV - 1, N_DEV)
        right = lax.rem(my + 1, N_DEV)

        barrier = pltpu.get_barrier_semaphore()
        for nbr in (left, right):
            pl.semaphore_signal(
                barrier, inc=1,
                device_id=(nbr,), device_id_type=pl.DeviceIdType.MESH,
            )
        pl.semaphore_wait(barrier, 2)

        wqg[0, :, :] = wq_ref[:, :]
        wog[0, :, :] = wo_ref[:, :]

        def copy(src, dst, sem_idx, target):
            return pltpu.make_async_remote_copy(
                src_ref=src, dst_ref=dst,
                send_sem=send_sems.at[sem_idx], recv_sem=recv_sems.at[sem_idx],
                device_id=(target,), device_id_type=pl.DeviceIdType.MESH,
            )

        rdmas = [
            copy(wqg.at[0], wqg.at[1], 0, right),
            copy(wog.at[0], wog.at[1], 1, right),
            copy(wqg.at[0], wqg.at[2], 2, left),
            copy(wog.at[0], wog.at[2], 3, left),
        ]
        for r in rdmas:
            r.start()

        def compute_block(slot, j):
            q_all = jnp.dot(x_ref[:, :], wqg[slot],
                            preferred_element_type=jnp.float32)
            q16 = q_all.astype(jnp.bfloat16)
            for b in range(B_loc):
                for hh in range(H_loc):
                    q = q16[b * Sq:(b + 1) * Sq, hh * Dh:(hh + 1) * Dh]
                    off = ((j * H_loc + hh) * B_loc + b) * Sq
                    k = k_ref[pl.ds(off, Skv), :]
                    v = v_ref[pl.ds(off, Skv), :]
                    s = lax.dot_general(
                        q, k, (((1,), (1,)), ((), ())),
                        preferred_element_type=jnp.float32) * 0.125
                    m = jnp.max(s, axis=1, keepdims=True)
                    w = jnp.exp(s - m)
                    w = (w / jnp.sum(w, axis=1, keepdims=True)).astype(
                        jnp.bfloat16)
                    ctx_ref[b * Sq:(b + 1) * Sq, hh * Dh:(hh + 1) * Dh] = (
                        jnp.dot(w, v, preferred_element_type=jnp.float32)
                        .astype(jnp.bfloat16))
            part = jnp.dot(ctx_ref[:, :], wog[slot],
                           preferred_element_type=jnp.float32)
            if slot == 0:
                out_ref[:, :] = part
            else:
                out_ref[:, :] += part

        compute_block(0, my)

        rdmas[0].wait_recv()
        rdmas[1].wait_recv()
        fwd_r = [
            copy(wqg.at[1, pl.ds(0, hq2)], wqg.at[3, pl.ds(0, hq2)], 4, right),
            copy(wog.at[1, pl.ds(0, ho2)], wog.at[3, pl.ds(0, ho2)], 5, right),
        ]
        for r in fwd_r:
            r.start()
        compute_block(1, left)

        rdmas[2].wait_recv()
        rdmas[3].wait_recv()
        fwd_l = [
            copy(wqg.at[2, pl.ds(hq2, hq2)], wqg.at[3, pl.ds(hq2, hq2)], 6, left),
            copy(wog.at[2, pl.ds(ho2, ho2)], wog.at[3, pl.ds(ho2, ho2)], 7, left),
        ]
        for r in fwd_l:
            r.start()
        compute_block(2, right)

        for r in fwd_r + fwd_l:
            r.wait_recv()
        compute_block(3, lax.rem(my + 2, N_DEV))

        for r in rdmas + fwd_r + fwd_l:
            r.wait_send()

    out_flat = pl.pallas_call(
        body,
        out_shape=jax.ShapeDtypeStruct((B_loc * Sq, Eo), jnp.float32),
        in_specs=[pl.BlockSpec(memory_space=pltpu.VMEM)] * 5,
        out_specs=pl.BlockSpec(memory_space=pltpu.VMEM),
        scratch_shapes=[
            pltpu.VMEM((N_DEV, E, wq_cols), jnp.bfloat16),
            pltpu.VMEM((N_DEV, wq_cols, Eo), jnp.bfloat16),
            pltpu.VMEM((B_loc * Sq, wq_cols), jnp.bfloat16),
            pltpu.SemaphoreType.DMA((8,)),
            pltpu.SemaphoreType.DMA((8,)),
        ],
        compiler_params=pltpu.CompilerParams(collective_id=0),
    )(xf, Wq16, Kb, Vb, Wo16)

    return out_flat.reshape(B_loc, Sq, Eo)
